# baseline (speedup 1.0000x reference)
import jax
import jax.numpy as jnp
from jax.experimental import pallas as pl

_NUM_CB = 4


def _vq_block(z_ref, cb_ref, cbm2_ref, wsq_ref, zq_ref, idx_ref, loss_ref):
    zb = z_ref[...]
    NT, C = zb.shape
    ncb, K, dpc = cb_ref.shape
    acc = jnp.zeros((), jnp.float32)
    for i in range(ncb):
        zi = zb[:, dpc * i:dpc * (i + 1)]
        wsq = wsq_ref[i]
        zsq = jnp.sum(zi * zi, axis=1, keepdims=True)
        prod = jax.lax.dot_general(
            zi, cbm2_ref[i], (((1,), (1,)), ((), ())),
            preferred_element_type=jnp.float32,
            precision=jax.lax.Precision.DEFAULT)
        d = (zsq + wsq) + prod
        m = jnp.min(d, axis=1, keepdims=True)
        col = jax.lax.broadcasted_iota(jnp.int32, d.shape, 1)
        idx = jnp.min(jnp.where(d == m, col, K), axis=1, keepdims=True)
        onehot = (col == idx).astype(jnp.bfloat16)
        zq = jax.lax.dot_general(
            onehot, cb_ref[i], (((1,), (0,)), ((), ())),
            preferred_element_type=jnp.float32,
            precision=jax.lax.Precision.DEFAULT)
        zq_ref[:, dpc * i:dpc * (i + 1)] = zq
        idx_ref[:, i:i + 1] = idx
        acc = acc + jnp.sum(m)

    first = pl.program_id(0) == 0
    acc2 = acc.reshape(1, 1)

    @pl.when(first)
    def _():
        loss_ref[:, :] = acc2

    @pl.when(jnp.logical_not(first))
    def _():
        loss_ref[:, :] = loss_ref[:, :] + acc2


def kernel(z, codebooks):
    b, c, h, w = z.shape
    n = b * h * w
    ncb, K, dpc = codebooks.shape
    z2 = jnp.transpose(z, (0, 2, 3, 1)).reshape(n, c)
    cbm2 = codebooks * (-2.0)
    wsq = jnp.sum(codebooks * codebooks, axis=2)[:, None, :]
    NT = 1024 if n % 1024 == 0 else n
    grid = (n // NT,)
    zq2, idx2, loss_acc = pl.pallas_call(
        _vq_block,
        grid=grid,
        in_specs=[
            pl.BlockSpec((NT, c), lambda t: (t, 0)),
            pl.BlockSpec((ncb, K, dpc), lambda t: (0, 0, 0)),
            pl.BlockSpec((ncb, K, dpc), lambda t: (0, 0, 0)),
            pl.BlockSpec((ncb, 1, K), lambda t: (0, 0, 0)),
        ],
        out_specs=[
            pl.BlockSpec((NT, c), lambda t: (t, 0)),
            pl.BlockSpec((NT, ncb), lambda t: (t, 0)),
            pl.BlockSpec((1, 1), lambda t: (0, 0)),
        ],
        out_shape=[
            jax.ShapeDtypeStruct((n, c), jnp.float32),
            jax.ShapeDtypeStruct((n, ncb), jnp.int32),
            jax.ShapeDtypeStruct((1, 1), jnp.float32),
        ],
    )(z2, codebooks, cbm2, wsq)
    z_q = jnp.transpose(zq2.reshape(b, h, w, c), (0, 3, 1, 2))
    total_loss = loss_acc[0, 0] * (1.25 / (ncb * n * dpc))
    indices = tuple(idx2[:, i].reshape(b, h, w) for i in range(ncb))
    return (z_q, total_loss, *indices)


# speedup vs baseline: 1.2477x; 1.2477x over previous
import jax
import jax.numpy as jnp
from jax.experimental import pallas as pl

_NUM_CB = 4


def _vq_block(z_ref, cb_ref, cbm2_ref, wsq_ref, zq_ref, idx_ref, loss_ref):
    zb = z_ref[...]
    NT, C = zb.shape
    ncb, K, dpc = cb_ref.shape
    acc = jnp.zeros((), jnp.float32)
    for i in range(ncb):
        zi = zb[:, dpc * i:dpc * (i + 1)]
        wsq = wsq_ref[i]
        zsq = jnp.sum(zi * zi, axis=1, keepdims=True)
        zsq_r = jax.lax.transpose(zsq, (1, 0))
        prod = jax.lax.dot_general(
            cbm2_ref[i], zi, (((1,), (1,)), ((), ())),
            preferred_element_type=jnp.float32,
            precision=jax.lax.Precision.DEFAULT)
        d = (zsq_r + wsq) + prod
        m = jnp.min(d, axis=0, keepdims=True)
        row = jax.lax.broadcasted_iota(jnp.int32, d.shape, 0)
        idx = jnp.min(jnp.where(d == m, row, K), axis=0, keepdims=True)
        onehot = (row == idx).astype(jnp.bfloat16)
        zq = jax.lax.dot_general(
            onehot, cb_ref[i], (((0,), (0,)), ((), ())),
            preferred_element_type=jnp.float32,
            precision=jax.lax.Precision.DEFAULT)
        zq_ref[:, dpc * i:dpc * (i + 1)] = zq
        idx_ref[i:i + 1, :] = idx
        acc = acc + jnp.sum(m)

    first = pl.program_id(0) == 0
    acc2 = acc.reshape(1, 1)

    @pl.when(first)
    def _():
        loss_ref[:, :] = acc2

    @pl.when(jnp.logical_not(first))
    def _():
        loss_ref[:, :] = loss_ref[:, :] + acc2


def kernel(z, codebooks):
    b, c, h, w = z.shape
    n = b * h * w
    ncb, K, dpc = codebooks.shape
    z2 = jnp.transpose(z, (0, 2, 3, 1)).reshape(n, c)
    cbm2 = codebooks * (-2.0)
    wsq = jnp.sum(codebooks * codebooks, axis=2)[:, :, None]
    NT = 1024 if n % 1024 == 0 else n
    grid = (n // NT,)
    zq2, idx2, loss_acc = pl.pallas_call(
        _vq_block,
        grid=grid,
        in_specs=[
            pl.BlockSpec((NT, c), lambda t: (t, 0)),
            pl.BlockSpec((ncb, K, dpc), lambda t: (0, 0, 0)),
            pl.BlockSpec((ncb, K, dpc), lambda t: (0, 0, 0)),
            pl.BlockSpec((ncb, K, 1), lambda t: (0, 0, 0)),
        ],
        out_specs=[
            pl.BlockSpec((NT, c), lambda t: (t, 0)),
            pl.BlockSpec((ncb, NT), lambda t: (0, t)),
            pl.BlockSpec((1, 1), lambda t: (0, 0)),
        ],
        out_shape=[
            jax.ShapeDtypeStruct((n, c), jnp.float32),
            jax.ShapeDtypeStruct((ncb, n), jnp.int32),
            jax.ShapeDtypeStruct((1, 1), jnp.float32),
        ],
    )(z2, codebooks, cbm2, wsq)
    z_q = jnp.transpose(zq2.reshape(b, h, w, c), (0, 3, 1, 2))
    total_loss = loss_acc[0, 0] * (1.25 / (ncb * n * dpc))
    indices = tuple(idx2[i].reshape(b, h, w) for i in range(ncb))
    return (z_q, total_loss, *indices)


# speedup vs baseline: 1.4424x; 1.1560x over previous
import jax
import jax.numpy as jnp
from jax.experimental import pallas as pl

_NUM_CB = 4


def _vq_block(z_ref, cb_ref, cbm2_ref, wsq_ref, zq_ref, idx_ref, loss_ref):
    zb = z_ref[...]
    NT, C = zb.shape
    ncb, K, dpc = cb_ref.shape
    acc = jnp.zeros((), jnp.float32)
    for i in range(ncb):
        zi = zb[:, dpc * i:dpc * (i + 1)]
        wsq = wsq_ref[i]
        zsq = jnp.sum(zi * zi, axis=1, keepdims=True)
        zsq_r = jax.lax.transpose(zsq, (1, 0))
        prod = jax.lax.dot_general(
            cbm2_ref[i], zi, (((1,), (1,)), ((), ())),
            preferred_element_type=jnp.float32,
            precision=jax.lax.Precision.DEFAULT)
        d = (zsq_r + wsq) + prod
        m = jnp.min(d, axis=0, keepdims=True)
        row = jax.lax.broadcasted_iota(jnp.int32, d.shape, 0)
        idx = jnp.min(jnp.where(d == m, row, K), axis=0, keepdims=True)
        idx_t = jax.lax.transpose(idx, (1, 0))
        col = jax.lax.broadcasted_iota(jnp.int32, (NT, K), 1)
        onehot = (col == idx_t).astype(jnp.bfloat16)
        zq = jax.lax.dot_general(
            onehot, cb_ref[i], (((1,), (0,)), ((), ())),
            preferred_element_type=jnp.float32,
            precision=jax.lax.Precision.DEFAULT)
        zq_ref[:, dpc * i:dpc * (i + 1)] = zq
        idx_ref[i:i + 1, :] = idx
        acc = acc + jnp.sum(m)

    first = pl.program_id(0) == 0
    acc2 = acc.reshape(1, 1)

    @pl.when(first)
    def _():
        loss_ref[:, :] = acc2

    @pl.when(jnp.logical_not(first))
    def _():
        loss_ref[:, :] = loss_ref[:, :] + acc2


def kernel(z, codebooks):
    b, c, h, w = z.shape
    n = b * h * w
    ncb, K, dpc = codebooks.shape
    z2 = jnp.transpose(z, (0, 2, 3, 1)).reshape(n, c)
    cbm2 = codebooks * (-2.0)
    wsq = jnp.sum(codebooks * codebooks, axis=2)[:, :, None]
    NT = 1024 if n % 1024 == 0 else n
    grid = (n // NT,)
    zq2, idx2, loss_acc = pl.pallas_call(
        _vq_block,
        grid=grid,
        in_specs=[
            pl.BlockSpec((NT, c), lambda t: (t, 0)),
            pl.BlockSpec((ncb, K, dpc), lambda t: (0, 0, 0)),
            pl.BlockSpec((ncb, K, dpc), lambda t: (0, 0, 0)),
            pl.BlockSpec((ncb, K, 1), lambda t: (0, 0, 0)),
        ],
        out_specs=[
            pl.BlockSpec((NT, c), lambda t: (t, 0)),
            pl.BlockSpec((ncb, NT), lambda t: (0, t)),
            pl.BlockSpec((1, 1), lambda t: (0, 0)),
        ],
        out_shape=[
            jax.ShapeDtypeStruct((n, c), jnp.float32),
            jax.ShapeDtypeStruct((ncb, n), jnp.int32),
            jax.ShapeDtypeStruct((1, 1), jnp.float32),
        ],
    )(z2, codebooks, cbm2, wsq)
    z_q = jnp.transpose(zq2.reshape(b, h, w, c), (0, 3, 1, 2))
    total_loss = loss_acc[0, 0] * (1.25 / (ncb * n * dpc))
    indices = tuple(idx2[i].reshape(b, h, w) for i in range(ncb))
    return (z_q, total_loss, *indices)


# speedup vs baseline: 1.5654x; 1.0853x over previous
import jax
import jax.numpy as jnp
from jax.experimental import pallas as pl

_NUM_CB = 4


def _vq_block(z_ref, cb_ref, cbm2_ref, wsq_ref, zq_ref, idx_ref, loss_ref):
    zb = z_ref[...]
    NT, C = zb.shape
    ncb, K, dpc = cb_ref.shape
    acc = jnp.zeros((), jnp.float32)
    row = jax.lax.broadcasted_iota(jnp.int32, (K, NT), 0)
    col = jax.lax.broadcasted_iota(jnp.int32, (NT, K), 1)
    for i in range(ncb):
        zi = zb[:, dpc * i:dpc * (i + 1)]
        wsq = wsq_ref[i]
        zsq = jnp.sum(zi * zi, axis=1, keepdims=True)
        zsq_r = jax.lax.transpose(zsq, (1, 0))
        prod = jax.lax.dot_general(
            cbm2_ref[i], zi, (((1,), (1,)), ((), ())),
            preferred_element_type=jnp.float32,
            precision=jax.lax.Precision.DEFAULT)
        d = (zsq_r + wsq) + prod
        m = jnp.min(d, axis=0, keepdims=True)
        idx = jnp.min(jnp.where(d == m, row, K), axis=0, keepdims=True)
        idx_t = jax.lax.transpose(idx, (1, 0))
        onehot = (col == idx_t).astype(jnp.bfloat16)
        zq = jax.lax.dot_general(
            onehot, cb_ref[i], (((1,), (0,)), ((), ())),
            preferred_element_type=jnp.float32,
            precision=jax.lax.Precision.DEFAULT)
        zq_ref[:, dpc * i:dpc * (i + 1)] = zq
        idx_ref[i:i + 1, :] = idx
        acc = acc + jnp.sum(m)

    first = pl.program_id(0) == 0
    acc2 = acc.reshape(1, 1)

    @pl.when(first)
    def _():
        loss_ref[:, :] = acc2

    @pl.when(jnp.logical_not(first))
    def _():
        loss_ref[:, :] = loss_ref[:, :] + acc2


def kernel(z, codebooks):
    b, c, h, w = z.shape
    n = b * h * w
    ncb, K, dpc = codebooks.shape
    z2 = jnp.transpose(z, (0, 2, 3, 1)).reshape(n, c)
    cbm2 = codebooks * (-2.0)
    wsq = jnp.sum(codebooks * codebooks, axis=2)[:, :, None]
    NT = 2048 if n % 2048 == 0 else n
    grid = (n // NT,)
    zq2, idx2, loss_acc = pl.pallas_call(
        _vq_block,
        grid=grid,
        in_specs=[
            pl.BlockSpec((NT, c), lambda t: (t, 0)),
            pl.BlockSpec((ncb, K, dpc), lambda t: (0, 0, 0)),
            pl.BlockSpec((ncb, K, dpc), lambda t: (0, 0, 0)),
            pl.BlockSpec((ncb, K, 1), lambda t: (0, 0, 0)),
        ],
        out_specs=[
            pl.BlockSpec((NT, c), lambda t: (t, 0)),
            pl.BlockSpec((ncb, NT), lambda t: (0, t)),
            pl.BlockSpec((1, 1), lambda t: (0, 0)),
        ],
        out_shape=[
            jax.ShapeDtypeStruct((n, c), jnp.float32),
            jax.ShapeDtypeStruct((ncb, n), jnp.int32),
            jax.ShapeDtypeStruct((1, 1), jnp.float32),
        ],
    )(z2, codebooks, cbm2, wsq)
    z_q = jnp.transpose(zq2.reshape(b, h, w, c), (0, 3, 1, 2))
    total_loss = loss_acc[0, 0] * (1.25 / (ncb * n * dpc))
    indices = tuple(idx2[i].reshape(b, h, w) for i in range(ncb))
    return (z_q, total_loss, *indices)


# speedup vs baseline: 1.8041x; 1.1524x over previous
import jax
import jax.numpy as jnp
from jax.experimental import pallas as pl

_NUM_CB = 4


def _vq_block(z_ref, cb_ref, cbm2_ref, wsq_ref, zq_ref, idx_ref, loss_ref):
    zb = z_ref[0]
    C, NT = zb.shape
    ncb, K, dpc = cb_ref.shape
    acc = jnp.zeros((), jnp.float32)
    for i in range(ncb):
        zi = zb[dpc * i:dpc * (i + 1), :]
        wsq = wsq_ref[i]
        zsq = jnp.sum(zi * zi, axis=0, keepdims=True)
        prod = jax.lax.dot_general(
            cbm2_ref[i], zi, (((1,), (0,)), ((), ())),
            preferred_element_type=jnp.float32,
            precision=jax.lax.Precision.DEFAULT)
        d = (zsq + wsq) + prod
        m = jnp.min(d, axis=0, keepdims=True)
        row = jax.lax.broadcasted_iota(jnp.int32, d.shape, 0)
        idx = jnp.min(jnp.where(d == m, row, K), axis=0, keepdims=True)
        onehot = (row == idx).astype(jnp.bfloat16)
        zq = jax.lax.dot_general(
            cb_ref[i], onehot, (((0,), (0,)), ((), ())),
            preferred_element_type=jnp.float32,
            precision=jax.lax.Precision.DEFAULT)
        zq_ref[0, dpc * i:dpc * (i + 1), :] = zq
        idx_ref[i:i + 1, :] = idx
        acc = acc + jnp.sum(m)

    first = jnp.logical_and(pl.program_id(0) == 0, pl.program_id(1) == 0)
    acc2 = acc.reshape(1, 1)

    @pl.when(first)
    def _():
        loss_ref[:, :] = acc2

    @pl.when(jnp.logical_not(first))
    def _():
        loss_ref[:, :] = loss_ref[:, :] + acc2


def kernel(z, codebooks):
    b, c, h, w = z.shape
    n = h * w
    ncb, K, dpc = codebooks.shape
    z3 = z.reshape(b, c, n)
    cbm2 = codebooks * (-2.0)
    wsq = jnp.sum(codebooks * codebooks, axis=2)[:, :, None]
    NT = 2048 if n % 2048 == 0 else n
    tpb = n // NT
    grid = (b, tpb)
    zq3, idx2, loss_acc = pl.pallas_call(
        _vq_block,
        grid=grid,
        in_specs=[
            pl.BlockSpec((1, c, NT), lambda bi, ti: (bi, 0, ti)),
            pl.BlockSpec((ncb, K, dpc), lambda bi, ti: (0, 0, 0)),
            pl.BlockSpec((ncb, K, dpc), lambda bi, ti: (0, 0, 0)),
            pl.BlockSpec((ncb, K, 1), lambda bi, ti: (0, 0, 0)),
        ],
        out_specs=[
            pl.BlockSpec((1, c, NT), lambda bi, ti: (bi, 0, ti)),
            pl.BlockSpec((ncb, NT), lambda bi, ti: (0, bi * tpb + ti)),
            pl.BlockSpec((1, 1), lambda bi, ti: (0, 0)),
        ],
        out_shape=[
            jax.ShapeDtypeStruct((b, c, n), jnp.float32),
            jax.ShapeDtypeStruct((ncb, b * n), jnp.int32),
            jax.ShapeDtypeStruct((1, 1), jnp.float32),
        ],
    )(z3, codebooks, cbm2, wsq)
    z_q = zq3.reshape(b, c, h, w)
    total_loss = loss_acc[0, 0] * (1.25 / (ncb * b * n * dpc))
    indices = tuple(idx2[i].reshape(b, h, w) for i in range(ncb))
    return (z_q, total_loss, *indices)
